# Initial kernel scaffold; baseline (speedup 1.0000x reference)
import jax
import jax.numpy as jnp
from jax.experimental import pallas as pl


def kernel(x, W, b):
    raise NotImplementedError("write your pallas kernel here")


# speedup vs baseline: 1.6237x; 1.6237x over previous
import jax
import jax.numpy as jnp
from jax.experimental import pallas as pl
from jax.experimental.pallas import tpu as pltpu

_S = 0.1
_R2 = 0.02
_EPS = 1e-12


def _body(x_ref, wt_ref, b_ref, o_ref):
    z = jnp.dot(x_ref[...], wt_ref[...], preferred_element_type=jnp.float32)
    z = z - b_ref[...]
    n = z.shape[-1]
    t = jnp.sum(z, axis=-1, keepdims=True)
    zz = jnp.sum(z * z, axis=-1, keepdims=True)

    y1 = z + (jnp.clip(t, -_S, _S) - t) / n
    ok1 = jnp.sum(y1 * y1, axis=-1, keepdims=True) <= _R2

    znorm = jnp.sqrt(jnp.maximum(zz, _EPS))
    scale = jnp.minimum(1.0, jnp.sqrt(_R2) / znorm)
    y2 = z * scale
    ok2 = jnp.abs(t) * scale <= _S

    denom = jnp.maximum(n * zz - t * t, _EPS)
    c = jnp.sqrt(jnp.maximum(n * _R2 - _S * _S, 0.0) / denom)
    sp = jnp.sign(t) * _S
    y3 = c * z + (sp - c * t) / n

    o_ref[...] = jnp.where(ok1, y1, jnp.where(ok2, y2, y3))


def kernel(x, W, b):
    B, D_in = x.shape
    D_out = W.shape[0]
    BT = 1024
    wt = W.T
    b2 = b.reshape(1, D_out)
    return pl.pallas_call(
        _body,
        grid=(B // BT,),
        in_specs=[
            pl.BlockSpec((BT, D_in), lambda i: (i, 0)),
            pl.BlockSpec((D_in, D_out), lambda i: (0, 0)),
            pl.BlockSpec((1, D_out), lambda i: (0, 0)),
        ],
        out_specs=pl.BlockSpec((BT, D_out), lambda i: (i, 0)),
        out_shape=jax.ShapeDtypeStruct((B, D_out), jnp.float32),
        compiler_params=pltpu.CompilerParams(
            dimension_semantics=("parallel",),
        ),
        name="optlayer_fused",
    )(x, wt, b2)


# speedup vs baseline: 1.6499x; 1.0161x over previous
import jax
import jax.numpy as jnp
from jax.experimental import pallas as pl
from jax.experimental.pallas import tpu as pltpu

_S = 0.1
_R2 = 0.02
_EPS = 1e-12


def _body(x_ref, wt_ref, b_ref, o_ref):
    z = jnp.dot(x_ref[...], wt_ref[...], preferred_element_type=jnp.float32)
    z = z - b_ref[...]
    n = z.shape[-1]
    t = jnp.sum(z, axis=-1, keepdims=True)
    zz = jnp.sum(z * z, axis=-1, keepdims=True)


    beta1 = (jnp.clip(t, -_S, _S) - t) / n
    ok1 = zz + (2.0 * t + n * beta1) * beta1 <= _R2

    znorm = jnp.sqrt(jnp.maximum(zz, _EPS))
    scale = jnp.minimum(1.0, jnp.sqrt(_R2) / znorm)
    ok2 = jnp.abs(t) * scale <= _S

    denom = jnp.maximum(n * zz - t * t, _EPS)
    c = jnp.sqrt(jnp.maximum(n * _R2 - _S * _S, 0.0) / denom)
    beta3 = (jnp.sign(t) * _S - c * t) / n

    alpha = jnp.where(ok1, 1.0, jnp.where(ok2, scale, c))
    beta = jnp.where(ok1, beta1, jnp.where(ok2, 0.0, beta3))
    o_ref[...] = alpha * z + beta


def kernel(x, W, b):
    B, D_in = x.shape
    D_out = W.shape[0]
    BT = 1024
    wt = W.T
    b2 = b.reshape(1, D_out)
    return pl.pallas_call(
        _body,
        grid=(B // BT,),
        in_specs=[
            pl.BlockSpec((BT, D_in), lambda i: (i, 0)),
            pl.BlockSpec((D_in, D_out), lambda i: (0, 0)),
            pl.BlockSpec((1, D_out), lambda i: (0, 0)),
        ],
        out_specs=pl.BlockSpec((BT, D_out), lambda i: (i, 0)),
        out_shape=jax.ShapeDtypeStruct((B, D_out), jnp.float32),
        compiler_params=pltpu.CompilerParams(
            dimension_semantics=("parallel",),
        ),
        name="optlayer_fused",
    )(x, wt, b2)


# speedup vs baseline: 2.0408x; 1.2369x over previous
import jax
import jax.numpy as jnp
from jax.experimental import pallas as pl
from jax.experimental.pallas import tpu as pltpu

_S = 0.1
_R2 = 0.02
_EPS = 1e-12


def _body(x_ref, wt_ref, b_ref, o_ref):
    z = jnp.dot(x_ref[...], wt_ref[...], preferred_element_type=jnp.float32)
    z = z - b_ref[...]
    n = z.shape[-1]
    t = jnp.sum(z, axis=-1, keepdims=True)
    zz = jnp.sum(z * z, axis=-1, keepdims=True)


    beta1 = (jnp.clip(t, -_S, _S) - t) / n
    ok1 = zz + (2.0 * t + n * beta1) * beta1 <= _R2

    znorm = jnp.sqrt(jnp.maximum(zz, _EPS))
    scale = jnp.minimum(1.0, jnp.sqrt(_R2) / znorm)
    ok2 = jnp.abs(t) * scale <= _S

    denom = jnp.maximum(n * zz - t * t, _EPS)
    c = jnp.sqrt(jnp.maximum(n * _R2 - _S * _S, 0.0) / denom)
    beta3 = (jnp.sign(t) * _S - c * t) / n

    alpha = jnp.where(ok1, 1.0, jnp.where(ok2, scale, c))
    beta = jnp.where(ok1, beta1, jnp.where(ok2, 0.0, beta3))
    o_ref[...] = alpha * z + beta


def kernel(x, W, b):
    B, D_in = x.shape
    D_out = W.shape[0]
    BT = 2048
    wt = W.T
    b2 = b.reshape(1, D_out)
    return pl.pallas_call(
        _body,
        grid=(B // BT,),
        in_specs=[
            pl.BlockSpec((BT, D_in), lambda i: (i, 0)),
            pl.BlockSpec((D_in, D_out), lambda i: (0, 0)),
            pl.BlockSpec((1, D_out), lambda i: (0, 0)),
        ],
        out_specs=pl.BlockSpec((BT, D_out), lambda i: (i, 0)),
        out_shape=jax.ShapeDtypeStruct((B, D_out), jnp.float32),
        compiler_params=pltpu.CompilerParams(
            dimension_semantics=("parallel",),
        ),
        name="optlayer_fused",
    )(x, wt, b2)


# speedup vs baseline: 2.2712x; 1.1129x over previous
import jax
import jax.numpy as jnp
from jax.experimental import pallas as pl
from jax.experimental.pallas import tpu as pltpu

_S = 0.1
_R2 = 0.02
_EPS = 1e-12


def _body(x_ref, wt_ref, b_ref, o_ref):
    z = jnp.dot(x_ref[...], wt_ref[...], preferred_element_type=jnp.float32)
    z = z - b_ref[...]
    n = z.shape[-1]
    t = jnp.sum(z, axis=-1, keepdims=True)
    zz = jnp.sum(z * z, axis=-1, keepdims=True)


    beta1 = (jnp.clip(t, -_S, _S) - t) / n
    ok1 = zz + (2.0 * t + n * beta1) * beta1 <= _R2

    znorm = jnp.sqrt(jnp.maximum(zz, _EPS))
    scale = jnp.minimum(1.0, jnp.sqrt(_R2) / znorm)
    ok2 = jnp.abs(t) * scale <= _S

    denom = jnp.maximum(n * zz - t * t, _EPS)
    c = jnp.sqrt(jnp.maximum(n * _R2 - _S * _S, 0.0) / denom)
    beta3 = (jnp.sign(t) * _S - c * t) / n

    alpha = jnp.where(ok1, 1.0, jnp.where(ok2, scale, c))
    beta = jnp.where(ok1, beta1, jnp.where(ok2, 0.0, beta3))
    o_ref[...] = alpha * z + beta


def kernel(x, W, b):
    B, D_in = x.shape
    D_out = W.shape[0]
    BT = 4096
    wt = W.T
    b2 = b.reshape(1, D_out)
    return pl.pallas_call(
        _body,
        grid=(B // BT,),
        in_specs=[
            pl.BlockSpec((BT, D_in), lambda i: (i, 0)),
            pl.BlockSpec((D_in, D_out), lambda i: (0, 0)),
            pl.BlockSpec((1, D_out), lambda i: (0, 0)),
        ],
        out_specs=pl.BlockSpec((BT, D_out), lambda i: (i, 0)),
        out_shape=jax.ShapeDtypeStruct((B, D_out), jnp.float32),
        compiler_params=pltpu.CompilerParams(
            dimension_semantics=("parallel",),
        ),
        name="optlayer_fused",
    )(x, wt, b2)


# speedup vs baseline: 2.3511x; 1.0352x over previous
import jax
import jax.numpy as jnp
from jax.experimental import pallas as pl
from jax.experimental.pallas import tpu as pltpu

_S = 0.1
_R2 = 0.02
_EPS = 1e-12


def _body(x_ref, wt_ref, b_ref, o_ref):
    z = jnp.dot(x_ref[...], wt_ref[...], preferred_element_type=jnp.float32)
    z = z - b_ref[...]
    n = z.shape[-1]
    t = jnp.sum(z, axis=-1, keepdims=True)
    zz = jnp.sum(z * z, axis=-1, keepdims=True)


    beta1 = (jnp.clip(t, -_S, _S) - t) / n
    ok1 = zz + (2.0 * t + n * beta1) * beta1 <= _R2

    znorm = jnp.sqrt(jnp.maximum(zz, _EPS))
    scale = jnp.minimum(1.0, jnp.sqrt(_R2) / znorm)
    ok2 = jnp.abs(t) * scale <= _S

    denom = jnp.maximum(n * zz - t * t, _EPS)
    c = jnp.sqrt(jnp.maximum(n * _R2 - _S * _S, 0.0) / denom)
    beta3 = (jnp.sign(t) * _S - c * t) / n

    alpha = jnp.where(ok1, 1.0, jnp.where(ok2, scale, c))
    beta = jnp.where(ok1, beta1, jnp.where(ok2, 0.0, beta3))
    o_ref[...] = alpha * z + beta


def kernel(x, W, b):
    B, D_in = x.shape
    D_out = W.shape[0]
    BT = 8192
    wt = W.T
    b2 = b.reshape(1, D_out)
    return pl.pallas_call(
        _body,
        grid=(B // BT,),
        in_specs=[
            pl.BlockSpec((BT, D_in), lambda i: (i, 0)),
            pl.BlockSpec((D_in, D_out), lambda i: (0, 0)),
            pl.BlockSpec((1, D_out), lambda i: (0, 0)),
        ],
        out_specs=pl.BlockSpec((BT, D_out), lambda i: (i, 0)),
        out_shape=jax.ShapeDtypeStruct((B, D_out), jnp.float32),
        compiler_params=pltpu.CompilerParams(
            dimension_semantics=("parallel",),
        ),
        name="optlayer_fused",
    )(x, wt, b2)


# speedup vs baseline: 2.6253x; 1.1166x over previous
import jax
import jax.numpy as jnp
from jax.experimental import pallas as pl
from jax.experimental.pallas import tpu as pltpu

_S = 0.1
_R2 = 0.02
_EPS = 1e-12


def _scalar_chain(t, zz, n):
    beta1 = (jnp.clip(t, -_S, _S) - t) * (1.0 / n)
    ok1 = zz + (2.0 * t + n * beta1) * beta1 <= _R2
    scale = jnp.minimum(1.0, jnp.sqrt(_R2) * jax.lax.rsqrt(jnp.maximum(zz, _EPS)))
    ok2 = jnp.abs(t) * scale <= _S
    denom = jnp.maximum(n * zz - t * t, _EPS)
    c = jnp.sqrt(jnp.maximum(n * _R2 - _S * _S, 0.0)) * jax.lax.rsqrt(denom)
    beta3 = (jnp.sign(t) * _S - c * t) * (1.0 / n)
    alpha = jnp.where(ok1, 1.0, jnp.where(ok2, scale, c))
    beta = jnp.where(ok1, beta1, jnp.where(ok2, 0.0, beta3))
    return alpha, beta


def _body(x_ref, w_ref, b_ref, o_ref):
    zt = jax.lax.dot_general(
        w_ref[...], x_ref[...],
        dimension_numbers=(((1,), (1,)), ((), ())),
        preferred_element_type=jnp.float32,
    )
    zt = zt - b_ref[...]
    n = zt.shape[0]
    t = jnp.sum(zt, axis=0, keepdims=True)
    zz = jnp.sum(zt * zt, axis=0, keepdims=True)
    alpha, beta = _scalar_chain(t, zz, n)
    o_ref[...] = (alpha * zt + beta).T


def kernel(x, W, b):
    B, D_in = x.shape
    D_out = W.shape[0]
    BT = 4096
    b2 = b.reshape(D_out, 1)
    return pl.pallas_call(
        _body,
        grid=(B // BT,),
        in_specs=[
            pl.BlockSpec((BT, D_in), lambda i: (i, 0)),
            pl.BlockSpec((D_out, D_in), lambda i: (0, 0)),
            pl.BlockSpec((D_out, 1), lambda i: (0, 0)),
        ],
        out_specs=pl.BlockSpec((BT, D_out), lambda i: (i, 0)),
        out_shape=jax.ShapeDtypeStruct((B, D_out), jnp.float32),
        compiler_params=pltpu.CompilerParams(
            dimension_semantics=("arbitrary",),
        ),
        name="optlayer_fused",
    )(x, W, b2)


# speedup vs baseline: 2.8205x; 1.0744x over previous
import jax
import jax.numpy as jnp
from jax.experimental import pallas as pl
from jax.experimental.pallas import tpu as pltpu

_S = 0.1
_R2 = 0.02
_EPS = 1e-12


def _scalar_chain(t, zz, n):
    beta1 = (jnp.clip(t, -_S, _S) - t) * (1.0 / n)
    ok1 = zz + (2.0 * t + n * beta1) * beta1 <= _R2
    scale = jnp.minimum(1.0, jnp.sqrt(_R2) * jax.lax.rsqrt(jnp.maximum(zz, _EPS)))
    ok2 = jnp.abs(t) * scale <= _S
    denom = jnp.maximum(n * zz - t * t, _EPS)
    c = jnp.sqrt(jnp.maximum(n * _R2 - _S * _S, 0.0)) * jax.lax.rsqrt(denom)
    beta3 = (jnp.sign(t) * _S - c * t) * (1.0 / n)
    alpha = jnp.where(ok1, 1.0, jnp.where(ok2, scale, c))
    beta = jnp.where(ok1, beta1, jnp.where(ok2, 0.0, beta3))
    return alpha, beta


def _body(x_ref, w_ref, b_ref, o_ref):
    zt = jax.lax.dot_general(
        w_ref[...], x_ref[...],
        dimension_numbers=(((1,), (1,)), ((), ())),
        preferred_element_type=jnp.float32,
    )
    zt = zt - b_ref[...]
    n = zt.shape[0]
    t = jnp.sum(zt, axis=0, keepdims=True)
    zz = jnp.sum(zt * zt, axis=0, keepdims=True)
    alpha, beta = _scalar_chain(t, zz, n)
    o_ref[...] = (alpha * zt + beta).T


def kernel(x, W, b):
    B, D_in = x.shape
    D_out = W.shape[0]
    BT = 8192
    b2 = b.reshape(D_out, 1)
    return pl.pallas_call(
        _body,
        grid=(B // BT,),
        in_specs=[
            pl.BlockSpec((BT, D_in), lambda i: (i, 0)),
            pl.BlockSpec((D_out, D_in), lambda i: (0, 0)),
            pl.BlockSpec((D_out, 1), lambda i: (0, 0)),
        ],
        out_specs=pl.BlockSpec((BT, D_out), lambda i: (i, 0)),
        out_shape=jax.ShapeDtypeStruct((B, D_out), jnp.float32),
        compiler_params=pltpu.CompilerParams(
            dimension_semantics=("arbitrary",),
        ),
        name="optlayer_fused",
    )(x, W, b2)
